# baseline (speedup 1.0000x reference)
import jax
import jax.numpy as jnp
from jax import lax
from jax.experimental import pallas as pl
from jax.experimental.pallas import tpu as pltpu

_B, _N, _C = 4, 256, 128
_O1, _O2 = 128, 64
_TN = 32
_NB = _N // _TN
_RB = 64
_G2 = (_B * _N) // _RB
_ROWS = _TN * _N
_M = _B * _N * _N
_KEEP = int(_N * (1.0 - 0.1))
_KDROP = _N - _KEEP
_EPS = 1e-5


def _outer_cols(a, b):
    return lax.dot_general(a, b, (((0,), (0,)), ((), ())),
                           preferred_element_type=jnp.float32)


def _stage1_body(vp_ref, w1_ref, g1_ref, bt1_ref, w2_ref,
                 y2_ref, nsim_ref, stats_ref, ab_ref):
    b = pl.program_id(0)
    j = pl.program_id(1)
    first = jnp.logical_and(b == 0, j == 0)

    @pl.when(first)
    def _prologue():
        sxx = jnp.zeros((_C, _C), jnp.float32)
        mx = jnp.zeros((1, _C), jnp.float32)
        for bb in range(_B):
            v = vp_ref[bb]
            v2 = v * v
            s1 = jnp.sum(v, axis=0, keepdims=True)
            s2 = jnp.sum(v2, axis=0, keepdims=True)
            dotc = lambda x, y: lax.dot_general(
                x, y, (((0,), (0,)), ((), ())),
                preferred_element_type=jnp.float32)
            p = dotc(v, v)
            r = dotc(v2, v2)
            vs = v * s1
            q1 = dotc(v2, vs)
            q2 = dotc(vs, v2)
            sxx = sxx + (2.0 * _N) * r + 2.0 * _outer_cols(s2, s2) \
                + 4.0 * p * p - 4.0 * q1 - 4.0 * q2
            mx = mx + (2.0 * _N) * s2 - 2.0 * (s1 * s1)
        inv_m = 1.0 / _M
        mean1 = lax.dot_general(w1_ref[...], mx, (((1,), (1,)), ((), ())),
                                preferred_element_type=jnp.float32)
        y = lax.dot_general(w1_ref[...], sxx, (((1,), (0,)), ((), ())),
                            preferred_element_type=jnp.float32)
        e2 = jnp.sum(y * w1_ref[...], axis=1, keepdims=True)
        mean1 = mean1 * inv_m
        var1 = e2 * inv_m - mean1 * mean1
        a1 = g1_ref[...] * lax.rsqrt(var1 + _EPS)
        b1 = bt1_ref[...] - mean1 * a1
        ab_col = jnp.concatenate([a1, b1], axis=1)
        eye = jnp.where(
            lax.broadcasted_iota(jnp.int32, (_O1, _O1), 0)
            == lax.broadcasted_iota(jnp.int32, (_O1, _O1), 1),
            1.0, 0.0).astype(jnp.float32)
        ab_ref[...] = lax.dot_general(
            ab_col, eye, (((0,), (0,)), ((), ())),
            preferred_element_type=jnp.float32)

    ab = ab_ref[...]
    a1 = ab[0:1, :]
    b1 = ab[1:2, :]
    vpb = vp_ref[b]
    vpn = vp_ref[b, pl.ds(j * _TN, _TN)]
    d = vpn[:, None, :] - vpb[None, :, :]
    x3 = d * d
    nsim_ref[...] = (-jnp.sum(x3, axis=2)).reshape(1, _TN, _N)
    x = x3.reshape(_ROWS, _C)
    y1 = lax.dot_general(x, w1_ref[...], (((1,), (1,)), ((), ())),
                         preferred_element_type=jnp.float32)
    h1 = y1 * a1 + b1
    h1 = jnp.where(h1 >= 0, h1, 0.01 * h1)
    y2 = lax.dot_general(h1, w2_ref[...], (((1,), (1,)), ((), ())),
                         preferred_element_type=jnp.float32)
    y2_ref[...] = y2.reshape(1, _ROWS, _O2)
    acc = jnp.concatenate(
        [jnp.sum(y2, axis=0, keepdims=True),
         jnp.sum(y2 * y2, axis=0, keepdims=True)], axis=0)

    @pl.when(first)
    def _init_stats():
        stats_ref[...] = acc

    @pl.when(jnp.logical_not(first))
    def _acc_stats():
        stats_ref[...] = stats_ref[...] + acc


def _stage2_body(y2_ref, stats_ref, g2_ref, bt2_ref, w3_ref, b3_ref,
                 ep_ref, out_ref):
    i = pl.program_id(0)
    inv_m = 1.0 / _M
    stats = stats_ref[...]
    mean2 = stats[0:1, :] * inv_m
    var2 = stats[1:2, :] * inv_m - mean2 * mean2
    a2 = g2_ref[...] * lax.rsqrt(var2 + _EPS)
    b2 = bt2_ref[...] - mean2 * a2

    y2 = y2_ref[0]
    h2 = y2 * a2 + b2
    h2 = jnp.where(h2 >= 0, h2, 0.01 * h2)
    y3 = jnp.sum(h2 * w3_ref[...], axis=1, keepdims=True) + b3_ref[...]
    sg = (1.0 / (1.0 + jnp.exp(-y3))).reshape(_RB, _N)

    rows = lax.broadcasted_iota(jnp.int32, (_RB, _N), 0)
    cols = lax.broadcasted_iota(jnp.int32, (_RB, _N), 1)
    diag = (i % (_N // _RB)) * _RB + rows
    is_diag = cols == diag
    epz = jnp.where(is_diag, 0.0, ep_ref[0])
    ep_sum = jnp.sum(epz, axis=1, keepdims=True)
    e = sg * epz

    bits = lax.bitcast_convert_type(e, jnp.int32)
    prefix = jnp.zeros((_RB, 1), jnp.int32)
    for bit in range(29, -1, -1):
        mid = prefix | jnp.int32(1 << bit)
        c = jnp.sum(jnp.where(bits < mid, 1.0, 0.0), axis=1, keepdims=True)
        prefix = jnp.where(c >= float(_KDROP), prefix, mid)
    cstar = jnp.sum(jnp.where(bits < prefix, 1, 0), axis=1, keepdims=True)
    eq = bits == prefix
    tri = jnp.where(
        lax.broadcasted_iota(jnp.int32, (_N, _N), 0)
        >= lax.broadcasted_iota(jnp.int32, (_N, _N), 1),
        1.0, 0.0).astype(jnp.float32)
    sfx = lax.dot_general(jnp.where(eq, 1.0, 0.0), tri,
                          (((1,), (0,)), ((), ())),
                          preferred_element_type=jnp.float32)
    dneed = (_KDROP - cstar).astype(jnp.float32)
    keep = (bits > prefix) | (eq & (sfx > dneed + 0.5))
    ek = jnp.where(keep, e, 0.0)
    l1 = jnp.maximum(jnp.sum(ek, axis=1, keepdims=True), 1e-12)
    out = ek * (ep_sum / l1)
    out = out + jnp.where(is_diag, 1.0, 0.0) + 1e-6
    out = out / jnp.sum(out, axis=1, keepdims=True)
    out_ref[...] = out.reshape(1, _RB, _N)


def kernel(vp_last_gen, ep_last_gen, W1, gamma1, beta1, W2, gamma2, beta2,
           W3, bias3):
    f32 = jnp.float32
    y2, nsim, stats = pl.pallas_call(
        _stage1_body,
        grid=(_B, _NB),
        in_specs=[
            pl.BlockSpec((_B, _N, _C), lambda b, j: (0, 0, 0)),
            pl.BlockSpec((_O1, _C), lambda b, j: (0, 0)),
            pl.BlockSpec((_O1, 1), lambda b, j: (0, 0)),
            pl.BlockSpec((_O1, 1), lambda b, j: (0, 0)),
            pl.BlockSpec((_O2, _C), lambda b, j: (0, 0)),
        ],
        out_specs=[
            pl.BlockSpec((1, _ROWS, _O2), lambda b, j: (b * _NB + j, 0, 0)),
            pl.BlockSpec((1, _TN, _N), lambda b, j: (b, j, 0)),
            pl.BlockSpec((2, _O2), lambda b, j: (0, 0)),
        ],
        out_shape=[
            jax.ShapeDtypeStruct((_B * _NB, _ROWS, _O2), f32),
            jax.ShapeDtypeStruct((_B, _N, _N), f32),
            jax.ShapeDtypeStruct((2, _O2), f32),
        ],
        scratch_shapes=[pltpu.VMEM((2, _O1), f32)],
    )(vp_last_gen, W1, gamma1.reshape(_O1, 1), beta1.reshape(_O1, 1), W2)

    ep_flat = ep_last_gen.reshape(_G2, _RB, _N)
    y2r = y2.reshape(_G2, _RB * _N, _O2)
    ep_out = pl.pallas_call(
        _stage2_body,
        grid=(_G2,),
        in_specs=[
            pl.BlockSpec((1, _RB * _N, _O2), lambda i: (i, 0, 0)),
            pl.BlockSpec((2, _O2), lambda i: (0, 0)),
            pl.BlockSpec((1, _O2), lambda i: (0, 0)),
            pl.BlockSpec((1, _O2), lambda i: (0, 0)),
            pl.BlockSpec((1, _O2), lambda i: (0, 0)),
            pl.BlockSpec((1, 1), lambda i: (0, 0)),
            pl.BlockSpec((1, _RB, _N), lambda i: (i, 0, 0)),
        ],
        out_specs=pl.BlockSpec((1, _RB, _N), lambda i: (i, 0, 0)),
        out_shape=jax.ShapeDtypeStruct((_G2, _RB, _N), f32),
    )(y2r, stats, gamma2.reshape(1, _O2), beta2.reshape(1, _O2),
      W3.reshape(1, _O2), bias3.reshape(1, 1), ep_flat)

    return ep_out.reshape(_B, _N, _N), nsim


# speedup vs baseline: 6.4753x; 6.4753x over previous
import jax
import jax.numpy as jnp
from jax import lax
from jax.experimental import pallas as pl
from jax.experimental.pallas import tpu as pltpu

_B, _N, _C = 4, 256, 128
_O1, _O2 = 128, 64
_TN = 32
_NB = _N // _TN
_P = _TN * _N
_G1 = _B * _NB
_RB = 64
_G2 = (_B * _N) // _RB
_M = _B * _N * _N
_KEEP = int(_N * (1.0 - 0.1))
_KDROP = _N - _KEEP
_EPS = 1e-5


def _dotl(a, b):
    return lax.dot_general(a, b, (((1,), (1,)), ((), ())),
                           preferred_element_type=jnp.float32)


def _mm(a, b):
    return lax.dot_general(a, b, (((1,), (0,)), ((), ())),
                           preferred_element_type=jnp.float32)


def _stage1_body(vpt_ref, vpn_ref, w1_ref, g1_ref, bt1_ref, w2_ref,
                 y2_ref, nsim_ref, stats_ref, ab_ref):
    b = pl.program_id(0)
    j = pl.program_id(1)
    first = jnp.logical_and(b == 0, j == 0)

    @pl.when(first)
    def _prologue():
        sxx = jnp.zeros((_C, _C), jnp.float32)
        mx = jnp.zeros((_C, 1), jnp.float32)
        for bb in range(_B):
            v = vpt_ref[bb]
            v2 = v * v
            s1 = jnp.sum(v, axis=1, keepdims=True)
            s2 = jnp.sum(v2, axis=1, keepdims=True)
            p = _dotl(v, v)
            r = _dotl(v2, v2)
            vs = v * s1
            q1 = _dotl(v2, vs)
            q2 = _dotl(vs, v2)
            sxx = sxx + (2.0 * _N) * r + 2.0 * _dotl(s2, s2) \
                + 4.0 * p * p - 4.0 * q1 - 4.0 * q2
            mx = mx + (2.0 * _N) * s2 - 2.0 * (s1 * s1)
        inv_m = 1.0 / _M
        mean1 = _mm(w1_ref[...], mx)
        y = _mm(w1_ref[...], sxx)
        e2 = jnp.sum(y * w1_ref[...], axis=1, keepdims=True)
        mean1 = mean1 * inv_m
        var1 = e2 * inv_m - mean1 * mean1
        a1 = g1_ref[...] * lax.rsqrt(var1 + _EPS)
        b1 = bt1_ref[...] - mean1 * a1
        ab_ref[...] = jnp.concatenate([a1, b1], axis=1)

    ab = ab_ref[...]
    a1 = ab[:, 0:1]
    b1 = ab[:, 1:2]
    vpbt = vpt_ref[b]
    vpnt = vpn_ref[0, 0]
    d = vpnt[:, :, None] - vpbt[:, None, :]
    x3 = d * d
    xt = x3.reshape(_C, _P)
    ones_c = jnp.ones((1, _C), jnp.float32)
    nsim_ref[...] = (-_mm(ones_c, xt)).reshape(1, 1, _P)
    y1 = _mm(w1_ref[...], xt)
    h1 = y1 * a1 + b1
    h1 = jnp.where(h1 >= 0, h1, 0.01 * h1)
    y2 = _mm(w2_ref[...], h1)
    y2_ref[...] = y2.reshape(1, _O2, _P)
    acc = jnp.concatenate(
        [jnp.sum(y2, axis=1, keepdims=True),
         jnp.sum(y2 * y2, axis=1, keepdims=True)], axis=1)

    @pl.when(first)
    def _init_stats():
        stats_ref[...] = acc

    @pl.when(jnp.logical_not(first))
    def _acc_stats():
        stats_ref[...] = stats_ref[...] + acc


def _stage2a_body(y2_ref, stats_ref, g2_ref, bt2_ref, w3_ref, b3_ref,
                  sg_ref):
    inv_m = 1.0 / _M
    stats = stats_ref[...]
    mean2 = stats[:, 0:1] * inv_m
    var2 = stats[:, 1:2] * inv_m - mean2 * mean2
    a2 = g2_ref[...] * lax.rsqrt(var2 + _EPS)
    b2 = bt2_ref[...] - mean2 * a2

    y2 = y2_ref[0]
    h2 = y2 * a2 + b2
    h2 = jnp.where(h2 >= 0, h2, 0.01 * h2)
    y3 = jnp.sum(h2 * w3_ref[...], axis=0, keepdims=True) + b3_ref[...]
    sg_ref[...] = (1.0 / (1.0 + jnp.exp(-y3))).reshape(1, 1, _P)


def _stage2b_body(sg_ref, ep_ref, out_ref):
    i = pl.program_id(0)
    rows = lax.broadcasted_iota(jnp.int32, (_RB, _N), 0)
    cols = lax.broadcasted_iota(jnp.int32, (_RB, _N), 1)
    diag = (i % (_N // _RB)) * _RB + rows
    is_diag = cols == diag
    epz = jnp.where(is_diag, 0.0, ep_ref[0])
    ep_sum = jnp.sum(epz, axis=1, keepdims=True)
    e = sg_ref[0] * epz

    bits = lax.bitcast_convert_type(e, jnp.int32)
    prefix = jnp.zeros((_RB, 1), jnp.int32)
    for bit in range(29, -1, -1):
        mid = prefix | jnp.int32(1 << bit)
        c = jnp.sum(jnp.where(bits < mid, 1.0, 0.0), axis=1, keepdims=True)
        prefix = jnp.where(c >= float(_KDROP), prefix, mid)
    cstar = jnp.sum(jnp.where(bits < prefix, 1, 0), axis=1, keepdims=True)
    eq = bits == prefix
    tri = jnp.where(
        lax.broadcasted_iota(jnp.int32, (_N, _N), 0)
        >= lax.broadcasted_iota(jnp.int32, (_N, _N), 1),
        1.0, 0.0).astype(jnp.float32)
    sfx = _mm(jnp.where(eq, 1.0, 0.0), tri)
    dneed = (_KDROP - cstar).astype(jnp.float32)
    keep = (bits > prefix) | (eq & (sfx > dneed + 0.5))
    ek = jnp.where(keep, e, 0.0)
    l1 = jnp.maximum(jnp.sum(ek, axis=1, keepdims=True), 1e-12)
    out = ek * (ep_sum / l1)
    out = out + jnp.where(is_diag, 1.0, 0.0) + 1e-6
    out = out / jnp.sum(out, axis=1, keepdims=True)
    out_ref[...] = out.reshape(1, _RB, _N)


def kernel(vp_last_gen, ep_last_gen, W1, gamma1, beta1, W2, gamma2, beta2,
           W3, bias3):
    f32 = jnp.float32
    vpt = jnp.swapaxes(vp_last_gen, 1, 2)
    y2, nsim, stats = pl.pallas_call(
        _stage1_body,
        grid=(_B, _NB),
        in_specs=[
            pl.BlockSpec((_B, _C, _N), lambda b, j: (0, 0, 0)),
            pl.BlockSpec((1, 1, _C, _TN), lambda b, j: (b, j, 0, 0)),
            pl.BlockSpec((_O1, _C), lambda b, j: (0, 0)),
            pl.BlockSpec((_O1, 1), lambda b, j: (0, 0)),
            pl.BlockSpec((_O1, 1), lambda b, j: (0, 0)),
            pl.BlockSpec((_O2, _C), lambda b, j: (0, 0)),
        ],
        out_specs=[
            pl.BlockSpec((1, _O2, _P), lambda b, j: (b * _NB + j, 0, 0)),
            pl.BlockSpec((1, 1, _P), lambda b, j: (b * _NB + j, 0, 0)),
            pl.BlockSpec((_O2, 2), lambda b, j: (0, 0)),
        ],
        out_shape=[
            jax.ShapeDtypeStruct((_G1, _O2, _P), f32),
            jax.ShapeDtypeStruct((_G1, 1, _P), f32),
            jax.ShapeDtypeStruct((_O2, 2), f32),
        ],
        scratch_shapes=[pltpu.VMEM((_O1, 2), f32)],
    )(vpt, vpt.reshape(_B, _C, _NB, _TN).transpose(0, 2, 1, 3),
      W1, gamma1.reshape(_O1, 1), beta1.reshape(_O1, 1), W2)

    sg = pl.pallas_call(
        _stage2a_body,
        grid=(_G1,),
        in_specs=[
            pl.BlockSpec((1, _O2, _P), lambda i: (i, 0, 0)),
            pl.BlockSpec((_O2, 2), lambda i: (0, 0)),
            pl.BlockSpec((_O2, 1), lambda i: (0, 0)),
            pl.BlockSpec((_O2, 1), lambda i: (0, 0)),
            pl.BlockSpec((_O2, 1), lambda i: (0, 0)),
            pl.BlockSpec((1, 1), lambda i: (0, 0)),
        ],
        out_specs=pl.BlockSpec((1, 1, _P), lambda i: (i, 0, 0)),
        out_shape=jax.ShapeDtypeStruct((_G1, 1, _P), f32),
    )(y2, stats, gamma2.reshape(_O2, 1), beta2.reshape(_O2, 1),
      W3.reshape(_O2, 1), bias3.reshape(1, 1))

    ep_out = pl.pallas_call(
        _stage2b_body,
        grid=(_G2,),
        in_specs=[
            pl.BlockSpec((1, _RB, _N), lambda i: (i, 0, 0)),
            pl.BlockSpec((1, _RB, _N), lambda i: (i, 0, 0)),
        ],
        out_specs=pl.BlockSpec((1, _RB, _N), lambda i: (i, 0, 0)),
        out_shape=jax.ShapeDtypeStruct((_G2, _RB, _N), f32),
    )(sg.reshape(_G2, _RB, _N), ep_last_gen.reshape(_G2, _RB, _N))

    return ep_out.reshape(_B, _N, _N), nsim.reshape(_B, _N, _N)


# speedup vs baseline: 7.8005x; 1.2046x over previous
import jax
import jax.numpy as jnp
from jax import lax
from jax.experimental import pallas as pl
from jax.experimental.pallas import tpu as pltpu

_B, _N, _C = 4, 256, 128
_O1, _O2 = 128, 64
_TN = 32
_NB = _N // _TN
_P = _TN * _N
_G1 = _B * _NB
_RB = 1024
_G2 = (_B * _N) // _RB
_M = _B * _N * _N
_KEEP = int(_N * (1.0 - 0.1))
_KDROP = _N - _KEEP
_EPS = 1e-5


def _dotl(a, b):
    return lax.dot_general(a, b, (((1,), (1,)), ((), ())),
                           preferred_element_type=jnp.float32)


def _mm(a, b):
    return lax.dot_general(a, b, (((1,), (0,)), ((), ())),
                           preferred_element_type=jnp.float32)


def _stage1_body(vpt_ref, vpn_ref, w1_ref, g1_ref, bt1_ref, w2_ref,
                 y2_ref, nsim_ref, stats_ref, ab_ref):
    b = pl.program_id(0)
    j = pl.program_id(1)
    first = jnp.logical_and(b == 0, j == 0)

    @pl.when(first)
    def _prologue():
        sxx = jnp.zeros((_C, _C), jnp.float32)
        mx = jnp.zeros((_C, 1), jnp.float32)
        for bb in range(_B):
            v = vpt_ref[bb]
            v2 = v * v
            s1 = jnp.sum(v, axis=1, keepdims=True)
            s2 = jnp.sum(v2, axis=1, keepdims=True)
            p = _dotl(v, v)
            r = _dotl(v2, v2)
            vs = v * s1
            q1 = _dotl(v2, vs)
            q2 = _dotl(vs, v2)
            sxx = sxx + (2.0 * _N) * r + 2.0 * _dotl(s2, s2) \
                + 4.0 * p * p - 4.0 * q1 - 4.0 * q2
            mx = mx + (2.0 * _N) * s2 - 2.0 * (s1 * s1)
        inv_m = 1.0 / _M
        mean1 = _mm(w1_ref[...], mx)
        y = _mm(w1_ref[...], sxx)
        e2 = jnp.sum(y * w1_ref[...], axis=1, keepdims=True)
        mean1 = mean1 * inv_m
        var1 = e2 * inv_m - mean1 * mean1
        a1 = g1_ref[...] * lax.rsqrt(var1 + _EPS)
        b1 = bt1_ref[...] - mean1 * a1
        ab_ref[...] = jnp.concatenate([a1, b1], axis=1)

    ab = ab_ref[...]
    a1 = ab[:, 0:1]
    b1 = ab[:, 1:2]
    vpbt = vpt_ref[b]
    vpnt = vpn_ref[0, 0]
    d = vpnt[:, :, None] - vpbt[:, None, :]
    x3 = d * d
    xt = x3.reshape(_C, _P)
    ones_c = jnp.ones((1, _C), jnp.float32)
    nsim_ref[...] = (-_mm(ones_c, xt)).reshape(1, 1, _P)
    y1 = _mm(w1_ref[...], xt)
    h1 = y1 * a1 + b1
    h1 = jnp.where(h1 >= 0, h1, 0.01 * h1)
    y2 = _mm(w2_ref[...], h1)
    y2_ref[...] = y2.reshape(1, _O2, _P)
    acc = jnp.concatenate(
        [jnp.sum(y2, axis=1, keepdims=True),
         jnp.sum(y2 * y2, axis=1, keepdims=True)], axis=1)

    @pl.when(first)
    def _init_stats():
        stats_ref[...] = acc

    @pl.when(jnp.logical_not(first))
    def _acc_stats():
        stats_ref[...] = stats_ref[...] + acc


def _stage2a_body(y2_ref, stats_ref, g2_ref, bt2_ref, w3_ref, b3_ref,
                  sg_ref):
    inv_m = 1.0 / _M
    stats = stats_ref[...]
    mean2 = stats[:, 0:1] * inv_m
    var2 = stats[:, 1:2] * inv_m - mean2 * mean2
    a2 = g2_ref[...] * lax.rsqrt(var2 + _EPS)
    b2 = bt2_ref[...] - mean2 * a2

    y2 = y2_ref[0]
    h2 = y2 * a2 + b2
    h2 = jnp.where(h2 >= 0, h2, 0.01 * h2)
    y3 = jnp.sum(h2 * w3_ref[...], axis=0, keepdims=True) + b3_ref[...]
    sg_ref[...] = (1.0 / (1.0 + jnp.exp(-y3))).reshape(1, 1, _P)


def _stage2b_body(sg_ref, ep_ref, out_ref):
    i = pl.program_id(0)
    rows = lax.broadcasted_iota(jnp.int32, (_RB, _N), 0)
    cols = lax.broadcasted_iota(jnp.int32, (_RB, _N), 1)
    diag = (i * _RB + rows) % _N
    is_diag = cols == diag
    epz = jnp.where(is_diag, 0.0, ep_ref[0])
    ep_sum = jnp.sum(epz, axis=1, keepdims=True)
    e = sg_ref[0] * epz

    bits = lax.bitcast_convert_type(e, jnp.int32)
    prefix = jnp.zeros((_RB, 1), jnp.int32)
    for bit in range(29, -1, -1):
        mid = prefix | jnp.int32(1 << bit)
        c = jnp.sum(jnp.where(bits < mid, 1.0, 0.0), axis=1, keepdims=True)
        prefix = jnp.where(c >= float(_KDROP), prefix, mid)
    cstar = jnp.sum(jnp.where(bits < prefix, 1, 0), axis=1, keepdims=True)
    eq = bits == prefix
    tri = jnp.where(
        lax.broadcasted_iota(jnp.int32, (_N, _N), 0)
        >= lax.broadcasted_iota(jnp.int32, (_N, _N), 1),
        1.0, 0.0).astype(jnp.float32)
    sfx = _mm(jnp.where(eq, 1.0, 0.0), tri)
    dneed = (_KDROP - cstar).astype(jnp.float32)
    keep = (bits > prefix) | (eq & (sfx > dneed + 0.5))
    ek = jnp.where(keep, e, 0.0)
    l1 = jnp.maximum(jnp.sum(ek, axis=1, keepdims=True), 1e-12)
    out = ek * (ep_sum / l1)
    out = out + jnp.where(is_diag, 1.0, 0.0) + 1e-6
    out = out / jnp.sum(out, axis=1, keepdims=True)
    out_ref[...] = out.reshape(1, _RB, _N)


def kernel(vp_last_gen, ep_last_gen, W1, gamma1, beta1, W2, gamma2, beta2,
           W3, bias3):
    f32 = jnp.float32
    vpt = jnp.swapaxes(vp_last_gen, 1, 2)
    y2, nsim, stats = pl.pallas_call(
        _stage1_body,
        grid=(_B, _NB),
        in_specs=[
            pl.BlockSpec((_B, _C, _N), lambda b, j: (0, 0, 0)),
            pl.BlockSpec((1, 1, _C, _TN), lambda b, j: (b, j, 0, 0)),
            pl.BlockSpec((_O1, _C), lambda b, j: (0, 0)),
            pl.BlockSpec((_O1, 1), lambda b, j: (0, 0)),
            pl.BlockSpec((_O1, 1), lambda b, j: (0, 0)),
            pl.BlockSpec((_O2, _C), lambda b, j: (0, 0)),
        ],
        out_specs=[
            pl.BlockSpec((1, _O2, _P), lambda b, j: (b * _NB + j, 0, 0)),
            pl.BlockSpec((1, 1, _P), lambda b, j: (b * _NB + j, 0, 0)),
            pl.BlockSpec((_O2, 2), lambda b, j: (0, 0)),
        ],
        out_shape=[
            jax.ShapeDtypeStruct((_G1, _O2, _P), f32),
            jax.ShapeDtypeStruct((_G1, 1, _P), f32),
            jax.ShapeDtypeStruct((_O2, 2), f32),
        ],
        scratch_shapes=[pltpu.VMEM((_O1, 2), f32)],
    )(vpt, vpt.reshape(_B, _C, _NB, _TN).transpose(0, 2, 1, 3),
      W1, gamma1.reshape(_O1, 1), beta1.reshape(_O1, 1), W2)

    sg = pl.pallas_call(
        _stage2a_body,
        grid=(_G1,),
        in_specs=[
            pl.BlockSpec((1, _O2, _P), lambda i: (i, 0, 0)),
            pl.BlockSpec((_O2, 2), lambda i: (0, 0)),
            pl.BlockSpec((_O2, 1), lambda i: (0, 0)),
            pl.BlockSpec((_O2, 1), lambda i: (0, 0)),
            pl.BlockSpec((_O2, 1), lambda i: (0, 0)),
            pl.BlockSpec((1, 1), lambda i: (0, 0)),
        ],
        out_specs=pl.BlockSpec((1, 1, _P), lambda i: (i, 0, 0)),
        out_shape=jax.ShapeDtypeStruct((_G1, 1, _P), f32),
    )(y2, stats, gamma2.reshape(_O2, 1), beta2.reshape(_O2, 1),
      W3.reshape(_O2, 1), bias3.reshape(1, 1))

    ep_out = pl.pallas_call(
        _stage2b_body,
        grid=(_G2,),
        in_specs=[
            pl.BlockSpec((1, _RB, _N), lambda i: (i, 0, 0)),
            pl.BlockSpec((1, _RB, _N), lambda i: (i, 0, 0)),
        ],
        out_specs=pl.BlockSpec((1, _RB, _N), lambda i: (i, 0, 0)),
        out_shape=jax.ShapeDtypeStruct((_G2, _RB, _N), f32),
    )(sg.reshape(_G2, _RB, _N), ep_last_gen.reshape(_G2, _RB, _N))

    return ep_out.reshape(_B, _N, _N), nsim.reshape(_B, _N, _N)


# speedup vs baseline: 8.2264x; 1.0546x over previous
import functools

import jax
import jax.numpy as jnp
from jax import lax
from jax.experimental import pallas as pl
from jax.experimental.pallas import tpu as pltpu
from jax.experimental.pallas import tpu_sc as plsc

_B, _N, _C = 4, 256, 128
_O1, _O2 = 128, 64
_TN = 64
_NB = _N // _TN
_P = _TN * _N
_G1 = _B * _NB
_M = _B * _N * _N
_KEEP = int(_N * (1.0 - 0.1))
_KDROP = _N - _KEEP
_EPS = 1e-5


def _dotl(a, b):
    return lax.dot_general(a, b, (((1,), (1,)), ((), ())),
                           preferred_element_type=jnp.float32)


def _mm(a, b):
    return lax.dot_general(a, b, (((1,), (0,)), ((), ())),
                           preferred_element_type=jnp.float32)


def _stage1_body(vpt_ref, vpn_ref, w1_ref, g1_ref, bt1_ref, w2_ref,
                 y2_ref, nsim_ref, stats_ref, ab_ref, w1s_ref, exp_ref):
    b = pl.program_id(0)
    j = pl.program_id(1)
    first = jnp.logical_and(b == 0, j == 0)

    @pl.when(first)
    def _prologue():
        sxx = jnp.zeros((_C, _C), jnp.float32)
        mx = jnp.zeros((_C, 1), jnp.float32)
        for bb in range(_B):
            v = vpt_ref[bb]
            v2 = v * v
            s1 = jnp.sum(v, axis=1, keepdims=True)
            s2 = jnp.sum(v2, axis=1, keepdims=True)
            p = _dotl(v, v)
            r = _dotl(v2, v2)
            vs = v * s1
            q1 = _dotl(v2, vs)
            q2 = _dotl(vs, v2)
            sxx = sxx + (2.0 * _N) * r + 2.0 * _dotl(s2, s2) \
                + 4.0 * p * p - 4.0 * q1 - 4.0 * q2
            mx = mx + (2.0 * _N) * s2 - 2.0 * (s1 * s1)
        inv_m = 1.0 / _M
        mean1 = _mm(w1_ref[...], mx)
        y = _mm(w1_ref[...], sxx)
        e2 = jnp.sum(y * w1_ref[...], axis=1, keepdims=True)
        mean1 = mean1 * inv_m
        var1 = e2 * inv_m - mean1 * mean1
        a1 = g1_ref[...] * lax.rsqrt(var1 + _EPS)
        b1 = bt1_ref[...] - mean1 * a1
        ab_ref[...] = jnp.concatenate([a1, b1], axis=1)
        w1s_ref[...] = w1_ref[...] * a1
        exp_ref[...] = jnp.where(
            lax.shift_right_logical(
                lax.broadcasted_iota(jnp.int32, (_TN, _P), 1), 8)
            == lax.broadcasted_iota(jnp.int32, (_TN, _P), 0),
            1.0, 0.0).astype(jnp.float32)

    b1 = ab_ref[:, 1:2]
    vpbt = vpt_ref[b]
    vpnt = vpn_ref[0, 0]
    vpn_rep = _mm(vpnt, exp_ref[...])
    vpb_tile = jnp.concatenate([vpbt] * _TN, axis=1)
    d = vpn_rep - vpb_tile
    xt = d * d
    ones_c = jnp.ones((1, _C), jnp.float32)
    nsim_ref[...] = (-_mm(ones_c, xt)).reshape(1, 1, _P)
    y1 = _mm(w1s_ref[...], xt)
    h1 = y1 + b1
    h1 = jnp.maximum(h1, 0.01 * h1)
    y2 = _mm(w2_ref[...], h1)
    y2_ref[...] = y2.reshape(1, _O2, _P)
    acc = jnp.concatenate(
        [jnp.sum(y2, axis=1, keepdims=True),
         jnp.sum(y2 * y2, axis=1, keepdims=True)], axis=1)

    @pl.when(first)
    def _init_stats():
        stats_ref[...] = acc

    @pl.when(jnp.logical_not(first))
    def _acc_stats():
        stats_ref[...] = stats_ref[...] + acc


def _stage2a_body(y2_ref, stats_ref, g2_ref, bt2_ref, w3_ref, b3_ref,
                  sg_ref):
    inv_m = 1.0 / _M
    stats = stats_ref[...]
    mean2 = stats[:, 0:1] * inv_m
    var2 = stats[:, 1:2] * inv_m - mean2 * mean2
    a2 = g2_ref[...] * lax.rsqrt(var2 + _EPS)
    b2 = bt2_ref[...] - mean2 * a2

    y2 = y2_ref[0]
    h2 = y2 * a2 + b2
    h2 = jnp.maximum(h2, 0.01 * h2)
    y3 = _mm(w3_ref[...], h2) + b3_ref[...]
    sg_ref[...] = (1.0 / (1.0 + jnp.exp(-y3))).reshape(1, 1, _P)


_NSC, _NSS = 2, 16
_NW = _NSC * _NSS
_NG = (_B * _N) // 16
_GPW = _NG // _NW


def _sc2b_body(sgt_hbm, ept_hbm, outt_hbm, sg_v, ep_v, ek_v, out_v):
    wid = lax.axis_index("s") * _NSC + lax.axis_index("c")
    lane = lax.broadcasted_iota(jnp.int32, (16,), 0)
    zi = jnp.zeros((16,), jnp.int32)
    zf = jnp.zeros((16,), jnp.float32)
    for g in range(_GPW):
        gid = wid * _GPW + g
        pltpu.sync_copy(sgt_hbm.at[gid], sg_v)
        pltpu.sync_copy(ept_hbm.at[gid], ep_v)
        diagm = (gid % (_N // 16)) * 16 + lane

        def p1(mm, eps):
            for u in range(8):
                m = mm * 8 + u
                epz = jnp.where(diagm == m, 0.0, ep_v[m])
                ek_v[m] = sg_v[m] * epz
                eps = eps + epz
            return eps
        epsum = lax.fori_loop(0, _N // 8, p1, zf)

        prefix = zi
        for bit in range(29, -1, -1):
            mid = prefix | jnp.int32(1 << bit)

            def cb(mm, cnt, mid=mid):
                for u in range(8):
                    m = mm * 8 + u
                    bv = lax.bitcast_convert_type(ek_v[m], jnp.int32)
                    cnt = cnt + jnp.where(bv < mid, 1, 0)
                return cnt
            c = lax.fori_loop(0, _N // 8, cb, zi)
            prefix = jnp.where(c >= _KDROP, prefix, mid)

        def p3(mm, carry):
            cs, ee = carry
            for u in range(8):
                m = mm * 8 + u
                bv = lax.bitcast_convert_type(ek_v[m], jnp.int32)
                cs = cs + jnp.where(bv < prefix, 1, 0)
                ee = ee + jnp.where(bv == prefix, 1, 0)
            return cs, ee
        cstar, etot = lax.fori_loop(0, _N // 8, p3, (zi, zi))
        keep_lim = etot - (_KDROP - cstar)

        def p4(mm, carry):
            l1, ordv = carry
            for u in range(8):
                m = mm * 8 + u
                e = ek_v[m]
                bv = lax.bitcast_convert_type(e, jnp.int32)
                iseq = bv == prefix
                keep = (bv > prefix) | (iseq & (ordv < keep_lim))
                ekk = jnp.where(keep, e, 0.0)
                ek_v[m] = ekk
                l1 = l1 + ekk
                ordv = ordv + jnp.where(iseq, 1, 0)
            return l1, ordv
        l1, _ = lax.fori_loop(0, _N // 8, p4, (zf, zi))
        scale = epsum / jnp.maximum(l1, 1e-12)
        tot = l1 * scale + (1.0 + _N * 1e-6)
        rinv = 1.0 / tot
        sr = scale * rinv

        def p5(mm, acc):
            for u in range(8):
                m = mm * 8 + u
                out_v[m] = ek_v[m] * sr \
                    + jnp.where(diagm == m, rinv + 1e-6 * rinv, 1e-6 * rinv)
            return acc
        lax.fori_loop(0, _N // 8, p5, jnp.int32(0))
        pltpu.sync_copy(out_v, outt_hbm.at[gid])


def kernel(vp_last_gen, ep_last_gen, W1, gamma1, beta1, W2, gamma2, beta2,
           W3, bias3):
    f32 = jnp.float32
    vpt = jnp.swapaxes(vp_last_gen, 1, 2)
    y2, nsim, stats = pl.pallas_call(
        _stage1_body,
        grid=(_B, _NB),
        in_specs=[
            pl.BlockSpec((_B, _C, _N), lambda b, j: (0, 0, 0)),
            pl.BlockSpec((1, 1, _C, _TN), lambda b, j: (b, j, 0, 0)),
            pl.BlockSpec((_O1, _C), lambda b, j: (0, 0)),
            pl.BlockSpec((_O1, 1), lambda b, j: (0, 0)),
            pl.BlockSpec((_O1, 1), lambda b, j: (0, 0)),
            pl.BlockSpec((_O2, _C), lambda b, j: (0, 0)),
        ],
        out_specs=[
            pl.BlockSpec((1, _O2, _P), lambda b, j: (b * _NB + j, 0, 0)),
            pl.BlockSpec((1, 1, _P), lambda b, j: (b * _NB + j, 0, 0)),
            pl.BlockSpec((_O2, 2), lambda b, j: (0, 0)),
        ],
        out_shape=[
            jax.ShapeDtypeStruct((_G1, _O2, _P), f32),
            jax.ShapeDtypeStruct((_G1, 1, _P), f32),
            jax.ShapeDtypeStruct((_O2, 2), f32),
        ],
        scratch_shapes=[pltpu.VMEM((_O1, 2), f32),
                        pltpu.VMEM((_O1, _C), f32),
                        pltpu.VMEM((_TN, _P), f32)],
    )(vpt, vpt.reshape(_B, _C, _NB, _TN).transpose(0, 2, 1, 3),
      W1, gamma1.reshape(_O1, 1), beta1.reshape(_O1, 1), W2)

    sg = pl.pallas_call(
        _stage2a_body,
        grid=(_G1,),
        in_specs=[
            pl.BlockSpec((1, _O2, _P), lambda i: (i, 0, 0)),
            pl.BlockSpec((_O2, 2), lambda i: (0, 0)),
            pl.BlockSpec((_O2, 1), lambda i: (0, 0)),
            pl.BlockSpec((_O2, 1), lambda i: (0, 0)),
            pl.BlockSpec((1, _O2), lambda i: (0, 0)),
            pl.BlockSpec((1, 1), lambda i: (0, 0)),
        ],
        out_specs=pl.BlockSpec((1, 1, _P), lambda i: (i, 0, 0)),
        out_shape=jax.ShapeDtypeStruct((_G1, 1, _P), f32),
    )(y2, stats, gamma2.reshape(_O2, 1), beta2.reshape(_O2, 1),
      W3.reshape(1, _O2), bias3.reshape(1, 1))

    sc2b = functools.partial(
        pl.kernel,
        mesh=plsc.VectorSubcoreMesh(core_axis_name="c", subcore_axis_name="s"),
        out_type=jax.ShapeDtypeStruct((_NG, _N, 16), f32),
        scratch_types=[
            pltpu.VMEM((_N, 16), f32),
            pltpu.VMEM((_N, 16), f32),
            pltpu.VMEM((_N, 16), f32),
            pltpu.VMEM((_N, 16), f32),
        ],
    )(_sc2b_body)
    sgt = sg.reshape(_NG, 16, _N).transpose(0, 2, 1)
    ept = ep_last_gen.reshape(_NG, 16, _N).transpose(0, 2, 1)
    ept_out = sc2b(sgt, ept)
    ep_out = ept_out.transpose(0, 2, 1).reshape(_B, _N, _N)

    return ep_out, nsim.reshape(_B, _N, _N)
